# speedup vs baseline: 1.1207x; 1.1207x over previous
import functools

import jax
import jax.numpy as jnp
from jax.experimental import pallas as pl
from jax.experimental.pallas import tpu as pltpu

D_MODEL = 4096
N_EXP = 64
K = 8
BLK = 512


def _router_body(x_ref, w_ref, gs_ref, ti_ref, tw_ref):
    x = x_ref[...]
    w = w_ref[...]
    logits = jax.lax.dot_general(
        x, w,
        dimension_numbers=(((1,), (1,)), ((), ())),
        preferred_element_type=jnp.float32,
    )
    m = jnp.max(logits, axis=1, keepdims=True)
    e = jnp.exp(logits - m)
    s = jnp.sum(e, axis=1, keepdims=True)
    gs = e / s
    gs_ref[...] = gs

    iota = jax.lax.broadcasted_iota(jnp.int32, (BLK, N_EXP), 1)
    work = gs
    idxs = []
    vals = []
    for _ in range(K):
        mx = jnp.max(work, axis=1, keepdims=True)
        is_max = work == mx
        idx = jnp.min(jnp.where(is_max, iota, N_EXP), axis=1, keepdims=True)
        idxs.append(idx)
        vals.append(mx)
        work = jnp.where(iota == idx, -jnp.inf, work)
    top_i = jnp.concatenate(idxs, axis=1)
    top_w = jnp.concatenate(vals, axis=1)
    tw_ref[...] = top_w / (jnp.sum(top_w, axis=1, keepdims=True) + 1e-8)
    ti_ref[...] = top_i


@jax.jit
def kernel(x, W):
    B = x.shape[0]
    grid = (B // BLK,)
    gs, ti, tw = pl.pallas_call(
        _router_body,
        grid=grid,
        in_specs=[
            pl.BlockSpec((BLK, D_MODEL), lambda i: (i, 0)),
            pl.BlockSpec((N_EXP, D_MODEL), lambda i: (0, 0)),
        ],
        out_specs=[
            pl.BlockSpec((BLK, N_EXP), lambda i: (i, 0)),
            pl.BlockSpec((BLK, K), lambda i: (i, 0)),
            pl.BlockSpec((BLK, K), lambda i: (i, 0)),
        ],
        out_shape=[
            jax.ShapeDtypeStruct((B, N_EXP), jnp.float32),
            jax.ShapeDtypeStruct((B, K), jnp.int32),
            jax.ShapeDtypeStruct((B, K), jnp.float32),
        ],
        compiler_params=pltpu.CompilerParams(
            dimension_semantics=("arbitrary",),
        ),
    )(x, W)
    return gs, ti, tw


# speedup vs baseline: 1.2911x; 1.1521x over previous
import functools

import jax
import jax.numpy as jnp
from jax.experimental import pallas as pl
from jax.experimental.pallas import tpu as pltpu

D_MODEL = 4096
N_EXP = 64
K = 8
BLK = 1024


def _router_body(x_ref, w_ref, gs_ref, ti_ref, tw_ref):
    x = x_ref[...]
    w = w_ref[...]
    logits = jax.lax.dot_general(
        x, w,
        dimension_numbers=(((1,), (1,)), ((), ())),
        preferred_element_type=jnp.float32,
    )
    m = jnp.max(logits, axis=1, keepdims=True)
    e = jnp.exp(logits - m)
    s = jnp.sum(e, axis=1, keepdims=True)
    gs = e / s
    gs_ref[...] = gs

    iota = jax.lax.broadcasted_iota(jnp.int32, (BLK, N_EXP), 1)
    work = gs
    idxs = []
    vals = []
    for _ in range(K):
        mx = jnp.max(work, axis=1, keepdims=True)
        is_max = work == mx
        idx = jnp.min(jnp.where(is_max, iota, N_EXP), axis=1, keepdims=True)
        idxs.append(idx)
        vals.append(mx)
        work = jnp.where(iota == idx, -jnp.inf, work)
    top_i = jnp.concatenate(idxs, axis=1)
    top_w = jnp.concatenate(vals, axis=1)
    tw_ref[...] = top_w / (jnp.sum(top_w, axis=1, keepdims=True) + 1e-8)
    ti_ref[...] = top_i


@jax.jit
def kernel(x, W):
    B = x.shape[0]
    grid = (B // BLK,)
    gs, ti, tw = pl.pallas_call(
        _router_body,
        grid=grid,
        in_specs=[
            pl.BlockSpec((BLK, D_MODEL), lambda i: (i, 0)),
            pl.BlockSpec((N_EXP, D_MODEL), lambda i: (0, 0)),
        ],
        out_specs=[
            pl.BlockSpec((BLK, N_EXP), lambda i: (i, 0)),
            pl.BlockSpec((BLK, K), lambda i: (i, 0)),
            pl.BlockSpec((BLK, K), lambda i: (i, 0)),
        ],
        out_shape=[
            jax.ShapeDtypeStruct((B, N_EXP), jnp.float32),
            jax.ShapeDtypeStruct((B, K), jnp.int32),
            jax.ShapeDtypeStruct((B, K), jnp.float32),
        ],
        compiler_params=pltpu.CompilerParams(
            dimension_semantics=("arbitrary",),
        ),
    )(x, W)
    return gs, ti, tw
